# baseline (speedup 1.0000x reference)
import jax
import jax.numpy as jnp
from jax.experimental import pallas as pl
from jax.experimental.pallas import tpu as pltpu

ALPHA = 0.2
N = 2048
NFEAT = 256
NHID = 16
NHEADS = 8
NFINAL = 64
ATT_HID = 16
BR = 512
NB = N // BR
H8 = NHEADS * NHID


def _elu(x):
    return jnp.where(x > 0, x, jnp.exp(x) - 1.0)


def _att1_kernel(adj_ref, feat_ref, wc_ref, fm1_ref, fm2_ref,
                 out_ref, zmin_ref, mask_ref,
                 whe_s, r_s, qt_s, qat_s):
    i = pl.program_id(0)
    g = jax.lax.rem(i, 2)
    blk = jax.lax.div(i, 2)

    @pl.when(i == 0)
    def _projections():
        feat = feat_ref[...]
        for g in range(2):
            wh = jnp.dot(feat, wc_ref[g], preferred_element_type=jnp.float32)
            whe_s[g] = jnp.zeros((N, NHEADS * 32), dtype=jnp.bfloat16)
            for h in range(NHEADS):
                whe_s[g, :, 32 * h:32 * h + NHID] = wh[
                    :, NHID * h:NHID * (h + 1)].astype(jnp.bfloat16)
                whe_s[g, :, 32 * h + NHID:32 * h + NHID + 1] = jnp.ones(
                    (N, 1), dtype=jnp.bfloat16)
            f1 = jnp.dot(wh, fm1_ref[g], preferred_element_type=jnp.float32)
            f2r = jnp.dot(wh, fm2_ref[g], preferred_element_type=jnp.float32)
            f2t = jax.lax.dot_general(fm2_ref[g], wh,
                                      (((0,), (1,)), ((), ())),
                                      preferred_element_type=jnp.float32)
            m2c = jnp.max(f2t, axis=1, keepdims=True)
            m2r = jnp.max(f2r, axis=0, keepdims=True)
            r_s[g] = jnp.exp(-(1.0 - ALPHA) * (f1 + m2r)).astype(jnp.bfloat16)
            qt_s[g] = jnp.exp(f2t - m2c).astype(jnp.bfloat16)
            qat_s[g] = jnp.exp(ALPHA * (f2t - m2c)).astype(jnp.bfloat16)

    zb = jnp.min(adj_ref[0], axis=(0, 1), keepdims=True)
    zmin_ref[0] = zb
    zmin = zb[0, 0]

    def heads(masked):
        if masked:
            mask = adj_ref[0] > 0
            mask_ref[0] = mask.astype(jnp.int8)
        for h in range(NHEADS):
            b = jnp.maximum(
                qt_s[g, h:h + 1, :],
                r_s[g, pl.ds(blk * BR, BR), h:h + 1] * qat_s[g, h:h + 1, :])
            if masked:
                b = jnp.where(mask, b, jnp.bfloat16(0.0))
            nd = jnp.dot(b, whe_s[g, :, 32 * h:32 * h + NHID + 1],
                         preferred_element_type=jnp.float32)
            out_ref[0, :, h * NHID:(h + 1) * NHID] = _elu(
                nd[:, :NHID] / nd[:, NHID:NHID + 1]).astype(jnp.bfloat16)

    pl.when(zmin > 0.0)(lambda: heads(False))
    pl.when(zmin <= 0.0)(lambda: heads(True))


def _att2_kernel(zmin_ref, mask_ref, h_ref, wout_ref, a1_ref, a2_ref,
                 w1_ref, b1_ref, w2_ref, out_ref,
                 emb_s, whe_s, r_s, qt_s, qat_s):
    i = pl.program_id(0)
    g = jax.lax.rem(i, 2)
    blk = jax.lax.div(i, 2)

    @pl.when(i == 0)
    def _projections():
        for gg in range(2):
            wh = jnp.dot(h_ref[gg].astype(jnp.float32), wout_ref[gg],
                         preferred_element_type=jnp.float32)
            whe_s[gg] = jnp.concatenate(
                [wh, jnp.ones((N, 1), dtype=jnp.float32)],
                axis=1).astype(jnp.bfloat16)
            f1 = jnp.dot(wh, a1_ref[gg], preferred_element_type=jnp.float32)
            f2t = jax.lax.dot_general(a2_ref[gg], wh, (((0,), (1,)), ((), ())),
                                      preferred_element_type=jnp.float32)
            m2 = jnp.max(f2t, axis=1, keepdims=True)
            r_s[gg] = jnp.exp(-(1.0 - ALPHA) * (f1 + m2)).astype(jnp.bfloat16)
            qt_s[gg] = jnp.exp(f2t - m2).astype(jnp.bfloat16)
            qat_s[gg] = jnp.exp(ALPHA * (f2t - m2)).astype(jnp.bfloat16)

    zmin = zmin_ref[0, 0, 0]

    def body(masked):
        b = jnp.maximum(qt_s[g],
                        r_s[g, pl.ds(blk * BR, BR), :] * qat_s[g])
        if masked:
            b = b * mask_ref[0].astype(jnp.bfloat16)
        nd = jnp.dot(b, whe_s[g], preferred_element_type=jnp.float32)
        emb_s[g] = _elu(nd[:, :NFINAL] / nd[:, NFINAL:NFINAL + 1])

    pl.when(zmin > 0.0)(lambda: body(False))
    pl.when(zmin <= 0.0)(lambda: body(True))

    @pl.when(g == 1)
    def _fuse():
        e1 = emb_s[0]
        e2 = emb_s[1]
        w1 = w1_ref[...]
        b1 = b1_ref[...]
        w2 = w2_ref[...]
        s1 = jnp.dot(jnp.tanh(jnp.dot(e1, w1) + b1), w2,
                     preferred_element_type=jnp.float32)
        s2 = jnp.dot(jnp.tanh(jnp.dot(e2, w1) + b1), w2,
                     preferred_element_type=jnp.float32)
        mx = jnp.maximum(s1, s2)
        a1 = jnp.exp(s1 - mx)
        a2 = jnp.exp(s2 - mx)
        out_ref[...] = (a1 * e1 + a2 * e2) / (a1 + a2)


def _head_mats(gat_W, gat_a):
    wc = jnp.transpose(gat_W, (1, 0, 2)).reshape(NFEAT, H8)
    a1 = gat_a[:, :NHID, 0]
    a2 = gat_a[:, NHID:, 0]
    eye = jnp.eye(NHEADS, dtype=jnp.float32)
    fm1 = (a1[:, :, None] * eye[:, None, :]).reshape(H8, NHEADS)
    fm2 = (a2[:, :, None] * eye[:, None, :]).reshape(H8, NHEADS)
    return wc, fm1, fm2


def kernel(feature, adj, gat1_W, gat1_a, gat1_Wout, gat1_aout, gat2_W, gat2_a,
           gat2_Wout, gat2_aout, att_W1, att_b1, att_W2):
    wc1, fm11, fm21 = _head_mats(gat1_W, gat1_a)
    wc2, fm12, fm22 = _head_mats(gat2_W, gat2_a)
    wc = jnp.stack([wc1, wc2])
    fm1 = jnp.stack([fm11, fm12])
    fm2 = jnp.stack([fm21, fm22])

    a1_out = pl.pallas_call(
        _att1_kernel,
        grid=(2 * NB,),
        in_specs=[
            pl.BlockSpec((1, BR, N), lambda i: (i % 2, i // 2, 0)),
            pl.BlockSpec((N, NFEAT), lambda i: (0, 0)),
            pl.BlockSpec((2, NFEAT, H8), lambda i: (0, 0, 0)),
            pl.BlockSpec((2, H8, NHEADS), lambda i: (0, 0, 0)),
            pl.BlockSpec((2, H8, NHEADS), lambda i: (0, 0, 0)),
        ],
        out_specs=[
            pl.BlockSpec((1, BR, H8), lambda i: (i % 2, i // 2, 0)),
            pl.BlockSpec((1, 1, 1), lambda i: (i, 0, 0)),
            pl.BlockSpec((1, BR, N), lambda i: (i % 2, i // 2, 0)),
        ],
        out_shape=[
            jax.ShapeDtypeStruct((2, N, H8), jnp.bfloat16),
            jax.ShapeDtypeStruct((2 * NB, 1, 1), jnp.float32),
            jax.ShapeDtypeStruct((2, N, N), jnp.int8),
        ],
        scratch_shapes=[
            pltpu.VMEM((2, N, NHEADS * 32), jnp.bfloat16),
            pltpu.VMEM((2, N, NHEADS), jnp.bfloat16),
            pltpu.VMEM((2, NHEADS, N), jnp.bfloat16),
            pltpu.VMEM((2, NHEADS, N), jnp.bfloat16),
        ],
    )(adj, feature, wc, fm1, fm2)
    h_cat, zmins, mask8 = a1_out

    wout = jnp.stack([gat1_Wout, gat2_Wout])
    a1o = jnp.stack([gat1_aout[:NFINAL], gat2_aout[:NFINAL]])
    a2o = jnp.stack([gat1_aout[NFINAL:], gat2_aout[NFINAL:]])

    out = pl.pallas_call(
        _att2_kernel,
        grid=(2 * NB,),
        in_specs=[
            pl.BlockSpec((1, 1, 1), lambda i: (i, 0, 0)),
            pl.BlockSpec((1, BR, N), lambda i: (i % 2, i // 2, 0)),
            pl.BlockSpec((2, N, H8), lambda i: (0, 0, 0)),
            pl.BlockSpec((2, H8, NFINAL), lambda i: (0, 0, 0)),
            pl.BlockSpec((2, NFINAL, 1), lambda i: (0, 0, 0)),
            pl.BlockSpec((2, NFINAL, 1), lambda i: (0, 0, 0)),
            pl.BlockSpec((NFINAL, ATT_HID), lambda i: (0, 0)),
            pl.BlockSpec((1, ATT_HID), lambda i: (0, 0)),
            pl.BlockSpec((ATT_HID, 1), lambda i: (0, 0)),
        ],
        out_specs=pl.BlockSpec((BR, NFINAL), lambda i: (i // 2, 0)),
        out_shape=jax.ShapeDtypeStruct((N, NFINAL), jnp.float32),
        scratch_shapes=[
            pltpu.VMEM((2, BR, NFINAL), jnp.float32),
            pltpu.VMEM((2, N, NFINAL + 1), jnp.bfloat16),
            pltpu.VMEM((2, N, 1), jnp.bfloat16),
            pltpu.VMEM((2, 1, N), jnp.bfloat16),
            pltpu.VMEM((2, 1, N), jnp.bfloat16),
        ],
    )(zmins, mask8, h_cat, wout, a1o, a2o, att_W1,
      att_b1.reshape(1, ATT_HID), att_W2)

    return out


# speedup vs baseline: 1.0131x; 1.0131x over previous
import jax
import jax.numpy as jnp
from jax.experimental import pallas as pl
from jax.experimental.pallas import tpu as pltpu

ALPHA = 0.2
N = 2048
NFEAT = 256
NHID = 16
NHEADS = 8
NFINAL = 64
ATT_HID = 16
BR = 512
NB = N // BR
H8 = NHEADS * NHID


def _elu(x):
    return jnp.where(x > 0, x, jnp.exp(x) - 1.0)


def _att1_kernel(adj_ref, feat_ref, wc_ref, fm1_ref, fm2_ref,
                 out_ref, zmin_ref, mask_ref,
                 whe_s, r_s, qt_s, qat_s):
    i = pl.program_id(0)
    g = jax.lax.rem(i, 2)
    blk = jax.lax.div(i, 2)

    @pl.when(i == 0)
    def _projections():
        feat = feat_ref[...]
        for g in range(2):
            wh = jnp.dot(feat, wc_ref[g], preferred_element_type=jnp.float32)
            whe_s[g] = jnp.zeros((N, NHEADS * 32), dtype=jnp.bfloat16)
            for h in range(NHEADS):
                whe_s[g, :, 32 * h:32 * h + NHID] = wh[
                    :, NHID * h:NHID * (h + 1)].astype(jnp.bfloat16)
                whe_s[g, :, 32 * h + NHID:32 * h + NHID + 1] = jnp.ones(
                    (N, 1), dtype=jnp.bfloat16)
            f1 = jnp.dot(wh, fm1_ref[g], preferred_element_type=jnp.float32)
            f2r = jnp.dot(wh, fm2_ref[g], preferred_element_type=jnp.float32)
            f2t = jax.lax.dot_general(fm2_ref[g], wh,
                                      (((0,), (1,)), ((), ())),
                                      preferred_element_type=jnp.float32)
            m2c = jnp.max(f2t, axis=1, keepdims=True)
            m2r = jnp.max(f2r, axis=0, keepdims=True)
            r_s[g] = jnp.exp(-(1.0 - ALPHA) * (f1 + m2r)).astype(jnp.bfloat16)
            qt_s[g] = jnp.exp(f2t - m2c).astype(jnp.bfloat16)
            qat_s[g] = jnp.exp(ALPHA * (f2t - m2c)).astype(jnp.bfloat16)

    zb = jnp.min(adj_ref[0], axis=(0, 1), keepdims=True)
    zmin_ref[0] = zb
    zmin = zb[0, 0]

    def heads(masked):
        if masked:
            mask = adj_ref[0] > 0
            mask_ref[0] = mask.astype(jnp.int8)
        for h in range(NHEADS):
            b = jnp.maximum(
                qt_s[g, h:h + 1, :],
                r_s[g, pl.ds(blk * BR, BR), h:h + 1] * qat_s[g, h:h + 1, :])
            if masked:
                b = jnp.where(mask, b, jnp.bfloat16(0.0))
            nd = jnp.dot(b, whe_s[g, :, 32 * h:32 * h + NHID + 1],
                         preferred_element_type=jnp.float32)
            out_ref[0, :, h * NHID:(h + 1) * NHID] = _elu(
                nd[:, :NHID] / nd[:, NHID:NHID + 1]).astype(jnp.bfloat16)

    pl.when(zmin > 0.0)(lambda: heads(False))
    pl.when(zmin <= 0.0)(lambda: heads(True))


def _att2_kernel(zmin_ref, mask_ref, h_ref, wout_ref, a1_ref, a2_ref,
                 w1_ref, b1_ref, w2_ref, out_ref,
                 emb_s, whe_s, r_s, qt_s, qat_s):
    i = pl.program_id(0)

    @pl.when(i == 0)
    def _projections():
        for gg in range(2):
            wh = jnp.dot(h_ref[gg].astype(jnp.float32), wout_ref[gg],
                         preferred_element_type=jnp.float32)
            whe_s[gg] = jnp.concatenate(
                [wh, jnp.ones((N, 1), dtype=jnp.float32)],
                axis=1).astype(jnp.bfloat16)
            f1 = jnp.dot(wh, a1_ref[gg], preferred_element_type=jnp.float32)
            f2t = jax.lax.dot_general(a2_ref[gg], wh, (((0,), (1,)), ((), ())),
                                      preferred_element_type=jnp.float32)
            m2 = jnp.max(f2t, axis=1, keepdims=True)
            r_s[gg] = jnp.exp(-(1.0 - ALPHA) * (f1 + m2)).astype(jnp.bfloat16)
            qt_s[gg] = jnp.exp(f2t - m2).astype(jnp.bfloat16)
            qat_s[gg] = jnp.exp(ALPHA * (f2t - m2)).astype(jnp.bfloat16)

    for g in range(2):
        zmin = zmin_ref[g, 0, 0]

        def body(masked, g=g):
            b = jnp.maximum(qt_s[g],
                            r_s[g, pl.ds(i * BR, BR), :] * qat_s[g])
            if masked:
                b = b * mask_ref[g].astype(jnp.bfloat16)
            nd = jnp.dot(b, whe_s[g], preferred_element_type=jnp.float32)
            emb_s[g] = _elu(nd[:, :NFINAL] / nd[:, NFINAL:NFINAL + 1])

        pl.when(zmin > 0.0)(lambda: body(False))
        pl.when(zmin <= 0.0)(lambda: body(True))

    e1 = emb_s[0]
    e2 = emb_s[1]
    w1 = w1_ref[...]
    b1 = b1_ref[...]
    w2 = w2_ref[...]
    s1 = jnp.dot(jnp.tanh(jnp.dot(e1, w1) + b1), w2,
                 preferred_element_type=jnp.float32)
    s2 = jnp.dot(jnp.tanh(jnp.dot(e2, w1) + b1), w2,
                 preferred_element_type=jnp.float32)
    mx = jnp.maximum(s1, s2)
    a1 = jnp.exp(s1 - mx)
    a2 = jnp.exp(s2 - mx)
    out_ref[...] = (a1 * e1 + a2 * e2) / (a1 + a2)


def _head_mats(gat_W, gat_a):
    wc = jnp.transpose(gat_W, (1, 0, 2)).reshape(NFEAT, H8)
    a1 = gat_a[:, :NHID, 0]
    a2 = gat_a[:, NHID:, 0]
    eye = jnp.eye(NHEADS, dtype=jnp.float32)
    fm1 = (a1[:, :, None] * eye[:, None, :]).reshape(H8, NHEADS)
    fm2 = (a2[:, :, None] * eye[:, None, :]).reshape(H8, NHEADS)
    return wc, fm1, fm2


def kernel(feature, adj, gat1_W, gat1_a, gat1_Wout, gat1_aout, gat2_W, gat2_a,
           gat2_Wout, gat2_aout, att_W1, att_b1, att_W2):
    wc1, fm11, fm21 = _head_mats(gat1_W, gat1_a)
    wc2, fm12, fm22 = _head_mats(gat2_W, gat2_a)
    wc = jnp.stack([wc1, wc2])
    fm1 = jnp.stack([fm11, fm12])
    fm2 = jnp.stack([fm21, fm22])

    a1_out = pl.pallas_call(
        _att1_kernel,
        grid=(2 * NB,),
        in_specs=[
            pl.BlockSpec((1, BR, N), lambda i: (i % 2, i // 2, 0)),
            pl.BlockSpec((N, NFEAT), lambda i: (0, 0)),
            pl.BlockSpec((2, NFEAT, H8), lambda i: (0, 0, 0)),
            pl.BlockSpec((2, H8, NHEADS), lambda i: (0, 0, 0)),
            pl.BlockSpec((2, H8, NHEADS), lambda i: (0, 0, 0)),
        ],
        out_specs=[
            pl.BlockSpec((1, BR, H8), lambda i: (i % 2, i // 2, 0)),
            pl.BlockSpec((1, 1, 1), lambda i: (i, 0, 0)),
            pl.BlockSpec((1, BR, N), lambda i: (i % 2, i // 2, 0)),
        ],
        out_shape=[
            jax.ShapeDtypeStruct((2, N, H8), jnp.bfloat16),
            jax.ShapeDtypeStruct((2 * NB, 1, 1), jnp.float32),
            jax.ShapeDtypeStruct((2, N, N), jnp.int8),
        ],
        scratch_shapes=[
            pltpu.VMEM((2, N, NHEADS * 32), jnp.bfloat16),
            pltpu.VMEM((2, N, NHEADS), jnp.bfloat16),
            pltpu.VMEM((2, NHEADS, N), jnp.bfloat16),
            pltpu.VMEM((2, NHEADS, N), jnp.bfloat16),
        ],
    )(adj, feature, wc, fm1, fm2)
    h_cat, zmins, mask8 = a1_out

    wout = jnp.stack([gat1_Wout, gat2_Wout])
    a1o = jnp.stack([gat1_aout[:NFINAL], gat2_aout[:NFINAL]])
    a2o = jnp.stack([gat1_aout[NFINAL:], gat2_aout[NFINAL:]])

    out = pl.pallas_call(
        _att2_kernel,
        grid=(NB,),
        in_specs=[
            pl.BlockSpec((2, 1, 1), lambda i: (i, 0, 0)),
            pl.BlockSpec((2, BR, N), lambda i: (0, i, 0)),
            pl.BlockSpec((2, N, H8), lambda i: (0, 0, 0)),
            pl.BlockSpec((2, H8, NFINAL), lambda i: (0, 0, 0)),
            pl.BlockSpec((2, NFINAL, 1), lambda i: (0, 0, 0)),
            pl.BlockSpec((2, NFINAL, 1), lambda i: (0, 0, 0)),
            pl.BlockSpec((NFINAL, ATT_HID), lambda i: (0, 0)),
            pl.BlockSpec((1, ATT_HID), lambda i: (0, 0)),
            pl.BlockSpec((ATT_HID, 1), lambda i: (0, 0)),
        ],
        out_specs=pl.BlockSpec((BR, NFINAL), lambda i: (i, 0)),
        out_shape=jax.ShapeDtypeStruct((N, NFINAL), jnp.float32),
        scratch_shapes=[
            pltpu.VMEM((2, BR, NFINAL), jnp.float32),
            pltpu.VMEM((2, N, NFINAL + 1), jnp.bfloat16),
            pltpu.VMEM((2, N, 1), jnp.bfloat16),
            pltpu.VMEM((2, 1, N), jnp.bfloat16),
            pltpu.VMEM((2, 1, N), jnp.bfloat16),
        ],
    )(zmins, mask8, h_cat, wout, a1o, a2o, att_W1,
      att_b1.reshape(1, ATT_HID), att_W2)

    return out
